# baseline (speedup 1.0000x reference)
import functools

import jax
import jax.numpy as jnp
from jax import lax
from jax.experimental import pallas as pl
from jax.experimental.pallas import tpu as pltpu
from jax.experimental.pallas import tpu_sc as plsc

N_NODES = 10000
D_FEAT = 128
N_EDGES = 320000

NC = 2
NS = 16
NW = NC * NS

CHUNK = 128
EPW = N_EDGES // NW
NCHUNK = 80
NSTAGE = 2
NCS = NCHUNK // NSTAGE
EPW_PAD = NCHUNK * CHUNK
NP_ROWS = 10112
RPS = NP_ROWS // NS


_mesh = plsc.VectorSubcoreMesh(core_axis_name="c", subcore_axis_name="s")


@functools.partial(
    pl.kernel,
    mesh=_mesh,
    out_type=jax.ShapeDtypeStruct((NC, NP_ROWS, D_FEAT), jnp.float32),
    scratch_types=[
        pltpu.VMEM((NCS, CHUNK), jnp.int32),
        pltpu.VMEM((NCS, CHUNK), jnp.int32),
        pltpu.VMEM((CHUNK, D_FEAT), jnp.float32),
        pltpu.VMEM((CHUNK, D_FEAT), jnp.float32),
        pltpu.VMEM_SHARED((NP_ROWS, D_FEAT), jnp.float32),
        pltpu.SemaphoreType.DMA,
        pltpu.SemaphoreType.DMA,
    ],
)
def _sc_propagate(src_hbm, dst_hbm, x_hbm, out_hbm,
                  src_v, dst_v, rows0, rows1, acc_sh, g0, g1):
    cid = lax.axis_index("c")
    sid = lax.axis_index("s")
    wid = sid * NC + cid

    @pl.loop(0, CHUNK)
    def _(r):
        @pl.loop(0, D_FEAT, step=16)
        def _(c):
            rows0[r, pl.ds(c, 16)] = jnp.zeros((16,), jnp.float32)

    base = sid * RPS

    @pl.loop(0, 4)
    def _(k):
        pltpu.sync_copy(rows0, acc_sh.at[pl.ds(base + k * CHUNK, CHUNK)])

    pltpu.sync_copy(rows0.at[pl.ds(0, RPS - 4 * CHUNK)],
                    acc_sh.at[pl.ds(base + 4 * CHUNK, RPS - 4 * CHUNK)])

    plsc.subcore_barrier()

    for st in range(NSTAGE):
        pltpu.sync_copy(src_hbm.at[wid].at[st], src_v)
        pltpu.sync_copy(dst_hbm.at[wid].at[st], dst_v)

        @pl.loop(0, NCS, step=2)
        def _(c):
            pltpu.async_copy(x_hbm.at[src_v.at[c]], rows0, g0)
            pltpu.async_copy(x_hbm.at[src_v.at[c + 1]], rows1, g1)
            pltpu.make_async_copy(x_hbm.at[src_v.at[c]], rows0, g0).wait()
            pltpu.sync_copy(rows0, acc_sh.at[dst_v.at[c]], add=True)
            pltpu.make_async_copy(x_hbm.at[src_v.at[c + 1]], rows1, g1).wait()
            pltpu.sync_copy(rows1, acc_sh.at[dst_v.at[c + 1]], add=True)

    plsc.subcore_barrier()

    pltpu.sync_copy(acc_sh.at[pl.ds(base, RPS)],
                    out_hbm.at[cid].at[pl.ds(base, RPS)])


def _combine_body(a_ref, b_ref, o_ref):
    o_ref[...] = a_ref[...] + b_ref[...]


def _combine(a, b):
    return pl.pallas_call(
        _combine_body,
        out_shape=jax.ShapeDtypeStruct((N_NODES, D_FEAT), jnp.float32),
        grid=(10,),
        in_specs=[pl.BlockSpec((N_NODES // 10, D_FEAT), lambda i: (i, 0)),
                  pl.BlockSpec((N_NODES // 10, D_FEAT), lambda i: (i, 0))],
        out_specs=pl.BlockSpec((N_NODES // 10, D_FEAT), lambda i: (i, 0)),
    )(a, b)


def kernel(edge_index, x):
    src = edge_index[0].reshape(NW, EPW)
    dst = edge_index[1].reshape(NW, EPW)
    pad = EPW_PAD - EPW
    src_p = jnp.pad(src, ((0, 0), (0, pad))).reshape(NW, NSTAGE, NCS, CHUNK)
    dst_p = jnp.pad(dst, ((0, 0), (0, pad)),
                    constant_values=N_NODES).reshape(NW, NSTAGE, NCS, CHUNK)
    partials = _sc_propagate(src_p, dst_p, x)
    return _combine(partials[0], partials[1])


# speedup vs baseline: 1.3889x; 1.3889x over previous
import functools

import jax
import jax.numpy as jnp
from jax import lax
from jax.experimental import pallas as pl
from jax.experimental.pallas import tpu as pltpu
from jax.experimental.pallas import tpu_sc as plsc

N_NODES = 10000
D_FEAT = 128
N_EDGES = 320000

NC = 2
NS = 16
NW = NC * NS

CHUNK = 128
EPW = N_EDGES // NW
NCHUNK = -(-EPW // CHUNK)
EPW_PAD = NCHUNK * CHUNK
NP_ROWS = 10112
RPS = NP_ROWS // NS
ORS = 632


_mesh = plsc.VectorSubcoreMesh(core_axis_name="c", subcore_axis_name="s")


@functools.partial(
    pl.kernel,
    mesh=_mesh,
    out_type=jax.ShapeDtypeStruct((NC, N_NODES, D_FEAT), jnp.float32),
    scratch_types=[
        pltpu.VMEM((NCHUNK, CHUNK), jnp.int32),
        pltpu.VMEM((NCHUNK, CHUNK), jnp.int32),
        pltpu.VMEM((CHUNK, D_FEAT), jnp.float32),
        pltpu.VMEM_SHARED((NP_ROWS, D_FEAT), jnp.float32),
        pltpu.SemaphoreType.DMA,
        pltpu.SemaphoreType.DMA,
        pltpu.SemaphoreType.DMA,
    ],
)
def _sc_propagate(src_hbm, dst_hbm, x_hbm, out_hbm,
                  src_v, dst_v, rows_v, acc_sh, sem, isem0, isem1):
    cid = lax.axis_index("c")
    sid = lax.axis_index("s")
    wid = sid * NC + cid

    pltpu.async_copy(src_hbm.at[wid], src_v, isem0)
    pltpu.async_copy(dst_hbm.at[wid], dst_v, isem1)

    @pl.loop(0, CHUNK)
    def _(r):
        @pl.loop(0, D_FEAT, step=16)
        def _(c):
            rows_v[r, pl.ds(c, 16)] = jnp.zeros((16,), jnp.float32)

    base = sid * RPS

    @pl.loop(0, 4)
    def _(k):
        pltpu.sync_copy(rows_v, acc_sh.at[pl.ds(base + k * CHUNK, CHUNK)])

    pltpu.sync_copy(rows_v.at[pl.ds(0, RPS - 4 * CHUNK)],
                    acc_sh.at[pl.ds(base + 4 * CHUNK, RPS - 4 * CHUNK)])

    pltpu.make_async_copy(src_hbm.at[wid], src_v, isem0).wait()
    pltpu.make_async_copy(dst_hbm.at[wid], dst_v, isem1).wait()

    plsc.subcore_barrier()

    @pl.loop(0, NCHUNK)
    def _(c):
        pltpu.async_copy(x_hbm.at[src_v.at[c]], rows_v, sem).wait()
        pltpu.sync_copy(rows_v, acc_sh.at[dst_v.at[c]], add=True)

    plsc.subcore_barrier()

    @pl.when(sid < NS - 1)
    def _():
        pltpu.sync_copy(acc_sh.at[pl.ds(base, ORS)],
                        out_hbm.at[cid].at[pl.ds(base, ORS)])

    @pl.when(sid == NS - 1)
    def _():
        pltpu.sync_copy(
            acc_sh.at[pl.ds((NS - 1) * ORS, N_NODES - (NS - 1) * ORS)],
            out_hbm.at[cid].at[pl.ds((NS - 1) * ORS, N_NODES - (NS - 1) * ORS)])


def _combine_body(a_ref, b_ref, o_ref):
    o_ref[...] = a_ref[...] + b_ref[...]


def _combine(a, b):
    return pl.pallas_call(
        _combine_body,
        out_shape=jax.ShapeDtypeStruct((N_NODES, D_FEAT), jnp.float32),
        grid=(10,),
        in_specs=[pl.BlockSpec((N_NODES // 10, D_FEAT), lambda i: (i, 0)),
                  pl.BlockSpec((N_NODES // 10, D_FEAT), lambda i: (i, 0))],
        out_specs=pl.BlockSpec((N_NODES // 10, D_FEAT), lambda i: (i, 0)),
    )(a, b)


def kernel(edge_index, x):
    src = edge_index[0].reshape(NW, EPW)
    dst = edge_index[1].reshape(NW, EPW)
    pad = EPW_PAD - EPW
    src_p = jnp.pad(src, ((0, 0), (0, pad))).reshape(NW, NCHUNK, CHUNK)
    dst_p = jnp.pad(dst, ((0, 0), (0, pad)),
                    constant_values=N_NODES).reshape(NW, NCHUNK, CHUNK)
    partials = _sc_propagate(src_p, dst_p, x)
    return _combine(partials[0], partials[1])
